# baseline (speedup 1.0000x reference)
import functools

import jax
import jax.numpy as jnp
from jax import lax
from jax.experimental import pallas as pl
from jax.experimental.pallas import tpu as pltpu
from jax.experimental.pallas import tpu_sc as plsc

_NUM_CORES = 2
_NUM_SUBCORES = 16
_NUM_WORKERS = _NUM_CORES * _NUM_SUBCORES
_LANES = 16


def _make_pooled_sum(seq, batch, dim):
    bpw = batch // _NUM_WORKERS
    mesh = plsc.VectorSubcoreMesh(core_axis_name="c", subcore_axis_name="s")

    @functools.partial(
        pl.kernel,
        mesh=mesh,
        out_type=jax.ShapeDtypeStruct((batch, dim), jnp.float32),
        compiler_params=pltpu.CompilerParams(use_tc_tiling_on_sc=False),
        scratch_types=[
            pltpu.VMEM((seq, bpw), jnp.int32),
            pltpu.VMEM((bpw, dim), jnp.float32),
            pltpu.VMEM((bpw, dim), jnp.float32),
            pltpu.VMEM((bpw, dim), jnp.float32),
            pltpu.SemaphoreType.DMA,
            pltpu.SemaphoreType.DMA,
        ],
    )
    def pooled_sum(text_hbm, table_hbm, out_hbm, idx_v, rows0, rows1, acc_v,
                   sem0, sem1):
        w = lax.axis_index("s") * _NUM_CORES + lax.axis_index("c")
        b0 = w * bpw

        pltpu.sync_copy(text_hbm.at[:, pl.ds(b0, bpw)], idx_v)

        def gather(s, buf, sem):
            return pltpu.make_async_copy(table_hbm.at[idx_v.at[s]], buf, sem)

        def accumulate(buf):
            @pl.loop(0, bpw)
            def _(i):
                for c in range(dim // _LANES):
                    sl = pl.ds(c * _LANES, _LANES)
                    acc_v[i, sl] = acc_v[i, sl] + buf[i, sl]

        @pl.loop(0, bpw)
        def _(i):
            for c in range(dim // _LANES):
                acc_v[i, pl.ds(c * _LANES, _LANES)] = jnp.zeros(
                    (_LANES,), jnp.float32)

        gather(0, rows0, sem0).start()

        @pl.loop(0, seq, step=2)
        def _(s):
            gather(s, rows0, sem0).wait()
            gather(s + 1, rows1, sem1).start()
            accumulate(rows0)
            gather(s + 1, rows1, sem1).wait()

            @pl.when(s + 2 < seq)
            def _():
                gather(s + 2, rows0, sem0).start()

            accumulate(rows1)

        pltpu.sync_copy(acc_v, out_hbm.at[pl.ds(b0, bpw)])

    return pooled_sum


def _fc_body(inv_seq, pooled_ref, wt_ref, b_ref, out_ref):
    x = pooled_ref[...] * inv_seq
    out_ref[...] = (
        jnp.dot(x, wt_ref[...], preferred_element_type=jnp.float32)
        + b_ref[...]
    )


def kernel(text, embedding_table, fc_w, fc_b):
    seq, batch = text.shape
    dim = embedding_table.shape[1]
    out_dim = fc_w.shape[0]

    pooled = _make_pooled_sum(seq, batch, dim)(text, embedding_table)

    fc = pl.pallas_call(
        functools.partial(_fc_body, 1.0 / seq),
        out_shape=jax.ShapeDtypeStruct((batch, out_dim), jnp.float32),
    )
    return fc(pooled, fc_w.T, fc_b.reshape(1, out_dim))
